# baseline (speedup 1.0000x reference)
import functools

import jax
import jax.numpy as jnp
from jax import lax
from jax.experimental import pallas as pl
from jax.experimental.pallas import tpu as pltpu
from jax.experimental.pallas import tpu_sc as plsc

_L = 16


def _make_kernel(B, S, D):
    mesh = plsc.VectorSubcoreMesh(core_axis_name="c", subcore_axis_name="s")

    @functools.partial(
        pl.kernel,
        out_type=jax.ShapeDtypeStruct((B, D), jnp.float32),
        mesh=mesh,
        scratch_types=[
            pltpu.VMEM((_L,), jnp.int32),
            pltpu.VMEM((_L, D), jnp.float32),
            pltpu.SemaphoreType.DMA,
        ],
    )
    def last_pool(x_hbm, leng_hbm, out_hbm, idx_v, rows_v, sem):
        wid = lax.axis_index("s") * 2 + lax.axis_index("c")

        @pl.when(wid == 0)
        def _():
            pltpu.sync_copy(leng_hbm, idx_v)
            lv = idx_v[...]
            lane = lax.iota(jnp.int32, _L)
            row = lax.rem(lv + (S - 1), S)
            flat = lane * S + row
            idx_v[...] = jnp.where(lane < B, flat, 0)
            pltpu.async_copy(x_hbm.at[idx_v], rows_v, sem).wait()
            pltpu.sync_copy(rows_v.at[pl.ds(0, B)], out_hbm)

    return last_pool


def kernel(x, leng):
    B, S, D = x.shape
    x_flat = x.reshape(B * S, D)
    leng_pad = jnp.concatenate(
        [leng.astype(jnp.int32), jnp.ones((_L - B,), jnp.int32)]
    )
    return _make_kernel(B, S, D)(x_flat, leng_pad)


# speedup vs baseline: 1.1491x; 1.1491x over previous
import functools

import jax
import jax.numpy as jnp
from jax import lax
from jax.experimental import pallas as pl
from jax.experimental.pallas import tpu as pltpu
from jax.experimental.pallas import tpu_sc as plsc

_L = 16


def _make_kernel(B, S, D):
    mesh = plsc.VectorSubcoreMesh(
        core_axis_name="c", subcore_axis_name="s", num_cores=1, num_subcores=1
    )

    @functools.partial(
        pl.kernel,
        out_type=jax.ShapeDtypeStruct((B, D), jnp.float32),
        mesh=mesh,
        scratch_types=[
            pltpu.VMEM((_L,), jnp.int32),
            pltpu.VMEM((B, D), jnp.float32),
            pltpu.SemaphoreType.DMA,
        ],
    )
    def last_pool(x_hbm, leng_hbm, out_hbm, idx_v, rows_v, sem):
        pltpu.sync_copy(leng_hbm, idx_v.at[pl.ds(0, B)])
        lv = idx_v[...]
        lane = lax.iota(jnp.int32, _L)
        row = lax.rem(lv + (S - 1), S)
        flat = lane * S + row
        idx_v[...] = jnp.where(lane < B, flat, 0)
        pltpu.async_copy(x_hbm.at[idx_v.at[pl.ds(0, B)]], rows_v, sem).wait()
        pltpu.sync_copy(rows_v, out_hbm)

    return last_pool


def kernel(x, leng):
    B, S, D = x.shape
    x_flat = x.reshape(B * S, D)
    return _make_kernel(B, S, D)(x_flat, leng.astype(jnp.int32))


# speedup vs baseline: 1.2561x; 1.0931x over previous
import functools

import jax
import jax.numpy as jnp
from jax import lax
from jax.experimental import pallas as pl
from jax.experimental.pallas import tpu as pltpu
from jax.experimental.pallas import tpu_sc as plsc


def _make_kernel(B, S, D):
    mesh = plsc.ScalarSubcoreMesh(axis_name="c", num_cores=1)

    @functools.partial(
        pl.kernel,
        out_type=jax.ShapeDtypeStruct((B, D), jnp.float32),
        mesh=mesh,
        scratch_types=[
            pltpu.SMEM((B,), jnp.int32),
            pltpu.SemaphoreType.DMA,
        ],
    )
    def last_pool(x_hbm, leng_hbm, out_hbm, leng_s, sem):
        pltpu.sync_copy(leng_hbm, leng_s)
        copies = []
        for b in range(B):
            idx = lax.rem(leng_s[b] + (S - 1), S)
            copies.append(
                pltpu.make_async_copy(x_hbm.at[b, idx], out_hbm.at[b], sem)
            )
        for c in copies:
            c.start()
        for c in copies:
            c.wait()

    return last_pool


def kernel(x, leng):
    B, S, D = x.shape
    return _make_kernel(B, S, D)(x, leng.astype(jnp.int32))
